# Initial kernel scaffold; baseline (speedup 1.0000x reference)
import jax
import jax.numpy as jnp
from jax.experimental import pallas as pl


def kernel(x, y, pos, spiral_idx_0, spiral_idx_1, spiral_idx_2, spiral_idx_3, dt0_row, dt0_col, dt0_val, dt1_row, dt1_col, dt1_val, dt2_row, dt2_col, dt2_val, dt3_row, dt3_col, dt3_val, ut0_row, ut0_col, ut0_val, ut1_row, ut1_col, ut1_val, ut2_row, ut2_col, ut2_val, ut3_row, ut3_col, ut3_val, en_w0, en_b0, en_w1, en_b1, en_w2, en_b2, en_w3, en_b3, en_lin_w, en_lin_b, reb_w, reb_b, rew_w, rew_b, reh_w, reh_b, cl_w, cl_b, lg_g_w, lg_g_b, lg_b_w, lg_b_b, lg_w_w, lg_w_b, lg_h_w, lg_h_b, de_lin_w, de_lin_b, de_w0, de_b0, de_w1, de_b1, de_w2, de_b2, de_w3, de_b3, de_final_w, de_final_b):
    raise NotImplementedError("write your pallas kernel here")


# speedup vs baseline: 3.1184x; 3.1184x over previous
import jax
import jax.numpy as jnp
from jax import lax
from jax.experimental import pallas as pl
from jax.experimental.pallas import tpu as pltpu
from jax.experimental.pallas import tpu_sc as plsc

_B = 16
_VS = (10000, 2500, 625, 157, 40)
_VP = (10000, 2504, 632, 160, 40)
_L = 9
_K = 3
_NC = 2
_NS = 16
_NW = _NC * _NS


def _sc_gather(table, idx_flat):
    N, D = table.shape
    M = idx_flat.shape[0]
    budget_rows = (384 * 1024) // (D * 4)
    sub = min(128, (budget_rows // 8) * 8)
    k_sub = max(1, min(6, budget_rows // sub))
    ch = k_sub * sub
    n_chunks = -(-M // ch)
    pad = n_chunks * ch - M
    if pad:
        idx_flat = jnp.pad(idx_flat, (0, pad))
    idx_nd = idx_flat.reshape(n_chunks, k_sub, sub)
    if k_sub < 8:
        idx_nd = jnp.pad(idx_nd, ((0, 0), (0, 8 - k_sub), (0, 0)))
    idx_scratch = pltpu.VMEM((8, sub), jnp.int32)
    g_full = M // ch
    tail = M - g_full * ch
    n_iter = -(-n_chunks // _NW)
    mesh = plsc.VectorSubcoreMesh(core_axis_name="c", subcore_axis_name="s")

    def body(table_ref, idx_ref, out_ref, idx_v, rows_v, sem):
        wid = lax.axis_index("s") * _NC + lax.axis_index("c")

        def run_chunk(g):
            pltpu.sync_copy(idx_ref.at[g], idx_v)
            cps = [
                pltpu.async_copy(
                    table_ref.at[idx_v.at[j]],
                    rows_v.at[pl.ds(j * sub, sub)],
                    sem,
                )
                for j in range(k_sub)
            ]
            for c in cps:
                c.wait()

        def step(it, carry):
            g = it * _NW + wid
            if g_full:
                @pl.when(g < g_full)
                def _():
                    run_chunk(g)
                    pltpu.sync_copy(rows_v, out_ref.at[pl.ds(g * ch, ch)])
            if tail:
                @pl.when(g == g_full)
                def _():
                    run_chunk(g)
                    pltpu.sync_copy(
                        rows_v.at[pl.ds(0, tail)],
                        out_ref.at[pl.ds(g_full * ch, tail)],
                    )
            return carry

        if n_iter > 1:
            lax.fori_loop(0, n_iter, step, 0)
        else:
            step(0, 0)

    return pl.kernel(
        body,
        out_type=jax.ShapeDtypeStruct((M, D), table.dtype),
        mesh=mesh,
        scratch_types=[
            idx_scratch,
            pltpu.VMEM((ch, D), jnp.float32),
            pltpu.SemaphoreType.DMA,
        ],
    )(table, idx_nd)


def _mm9(g9, w, bias, act):
    _, M, C = g9.shape
    N = w.shape[1]
    bm = 512
    grid = -(-M // bm)

    def body(g_ref, w_ref, b_ref, o_ref):
        acc = b_ref[...].astype(jnp.float32)
        for l in range(_L):
            acc = acc + jnp.dot(
                g_ref[l], w_ref[pl.ds(l * C, C), :],
                preferred_element_type=jnp.float32,
            )
        if act:
            acc = jnp.where(acc > 0, acc, jnp.exp(jnp.minimum(acc, 0.0)) - 1.0)
        o_ref[...] = acc

    return pl.pallas_call(
        body,
        grid=(grid,),
        in_specs=[
            pl.BlockSpec((_L, bm, C), lambda i: (0, i, 0)),
            pl.BlockSpec((_L * C, N), lambda i: (0, 0)),
            pl.BlockSpec((1, N), lambda i: (0, 0)),
        ],
        out_specs=pl.BlockSpec((bm, N), lambda i: (i, 0)),
        out_shape=jax.ShapeDtypeStruct((M, N), jnp.float32),
    )(g9, w, bias.reshape(1, N))


def _wsum(g, w):
    vo, d3 = g.shape
    D = d3 // _K
    bm = 256
    grid = -(-vo // bm)

    def body(g_ref, w_ref, o_ref):
        gg = g_ref[...]
        ww = w_ref[...]
        o_ref[...] = (
            ww[:, 0:1] * gg[:, :D]
            + ww[:, 1:2] * gg[:, D : 2 * D]
            + ww[:, 2:3] * gg[:, 2 * D :]
        )

    return pl.pallas_call(
        body,
        grid=(grid,),
        in_specs=[
            pl.BlockSpec((bm, d3), lambda i: (i, 0)),
            pl.BlockSpec((bm, _K), lambda i: (i, 0)),
        ],
        out_specs=pl.BlockSpec((bm, D), lambda i: (i, 0)),
        out_shape=jax.ShapeDtypeStruct((vo, D), jnp.float32),
    )(g, w)


def _middle(gz, w1, b1, wh, bh, w2, b2, in4, wl, bl):
    def body(gz_ref, w1_ref, b1_ref, wh_ref, bh_ref, w2_ref, b2_ref,
             i4_ref, wl_ref, bl_ref, z_ref, d0_ref, h4_ref, lg_ref):
        gzv = gz_ref[...]
        z = jnp.dot(gzv, w1_ref[...], preferred_element_type=jnp.float32) + b1_ref[...]
        z_ref[...] = z
        d0_ref[...] = (
            jnp.dot(z, w2_ref[...], preferred_element_type=jnp.float32) + b2_ref[...]
        )
        h4 = jnp.dot(gzv, wh_ref[...], preferred_element_type=jnp.float32) + bh_ref[...]
        is_cl = lax.broadcasted_iota(jnp.int32, h4.shape, 1) == 3
        h4_ref[...] = jnp.where(is_cl, 1.0 / (1.0 + jnp.exp(-h4)), h4)
        lg_ref[...] = (
            jnp.dot(i4_ref[...], wl_ref[...], preferred_element_type=jnp.float32)
            + bl_ref[...]
        )

    nv = gz.shape[1]
    return pl.pallas_call(
        body,
        out_shape=[
            jax.ShapeDtypeStruct((_B, w1.shape[1]), jnp.float32),
            jax.ShapeDtypeStruct((_B, nv), jnp.float32),
            jax.ShapeDtypeStruct((_B, 4), jnp.float32),
            jax.ShapeDtypeStruct((_B, 128), jnp.float32),
        ],
    )(gz, w1, b1.reshape(1, -1), wh, bh.reshape(1, -1), w2, b2.reshape(1, -1),
      in4, wl, bl.reshape(1, -1))


def _conv(h, c, sidx, w, bias, act):
    v = sidx.shape[0]
    g = _sc_gather(h, sidx.T.reshape(-1))
    g9 = g.reshape(_L, v * _B, c)
    return _mm9(g9, w, bias, act)


def _pool(h, c, col, val, vo):
    g = _sc_gather(h, col)
    return _wsum(g.reshape(vo, _K * _B * c), val.reshape(vo, _K))


def kernel(x, y, pos, spiral_idx_0, spiral_idx_1, spiral_idx_2, spiral_idx_3, dt0_row, dt0_col, dt0_val, dt1_row, dt1_col, dt1_val, dt2_row, dt2_col, dt2_val, dt3_row, dt3_col, dt3_val, ut0_row, ut0_col, ut0_val, ut1_row, ut1_col, ut1_val, ut2_row, ut2_col, ut2_val, ut3_row, ut3_col, ut3_val, en_w0, en_b0, en_w1, en_b1, en_w2, en_b2, en_w3, en_b3, en_lin_w, en_lin_b, reb_w, reb_b, rew_w, rew_b, reh_w, reh_b, cl_w, cl_b, lg_g_w, lg_g_b, lg_b_w, lg_b_b, lg_w_w, lg_w_b, lg_h_w, lg_h_b, de_lin_w, de_lin_b, de_w0, de_b0, de_w1, de_b1, de_w2, de_b2, de_w3, de_b3, de_final_w, de_final_b):
    def padv(a, n):
        return jnp.pad(a, ((0, n - a.shape[0]),) + ((0, 0),) * (a.ndim - 1))

    sidx = tuple(padv(s, _VP[i])
                 for i, s in enumerate((spiral_idx_0, spiral_idx_1, spiral_idx_2, spiral_idx_3)))
    dcol = tuple(padv(c, _VP[i + 1] * _K)
                 for i, c in enumerate((dt0_col, dt1_col, dt2_col, dt3_col)))
    dval = tuple(padv(v, _VP[i + 1] * _K)
                 for i, v in enumerate((dt0_val, dt1_val, dt2_val, dt3_val)))
    ucol = tuple(padv(c, _VP[i] * _K)
                 for i, c in enumerate((ut0_col, ut1_col, ut2_col, ut3_col)))
    uval = tuple(padv(v, _VP[i] * _K)
                 for i, v in enumerate((ut0_val, ut1_val, ut2_val, ut3_val)))
    en_w = (en_w0, en_w1, en_w2, en_w3)
    en_b = (en_b0, en_b1, en_b2, en_b3)
    de_w = (de_w0, de_w1, de_w2, de_w3)
    de_b = (de_b0, de_b1, de_b2, de_b3)
    chans = (8, 32, 32, 32)

    h = jnp.pad(x.transpose(1, 0, 2), ((0, 0), (0, 0), (0, 5))).reshape(_VS[0], _B * 8)
    w0 = jnp.pad(en_w0.reshape(_L, 3, -1), ((0, 0), (0, 5), (0, 0))).reshape(_L * 8, -1)
    for i in range(4):
        wi = w0 if i == 0 else en_w[i]
        co = en_w[i].shape[1]
        h = _conv(h, chans[i], sidx[i], wi, en_b[i], True)
        h = _pool(h.reshape(_VP[i], _B * co), co, dcol[i], dval[i], _VP[i + 1])

    gender_z = h.reshape(_VS[4], _B, 64).transpose(1, 0, 2).reshape(_B, _VS[4] * 64)

    wh = jnp.concatenate([reb_w, rew_w, reh_w, cl_w], axis=1)
    bh = jnp.concatenate([reb_b, rew_b, reh_b, cl_b])
    in4 = jnp.concatenate([y, pos[:, 0:1], pos[:, 1:2], pos[:, 2:3]], axis=1)
    wl = jnp.zeros((4, 128), jnp.float32)
    wl = wl.at[0, 0:32].set(lg_g_w[0]).at[1, 32:64].set(lg_b_w[0])
    wl = wl.at[2, 64:96].set(lg_w_w[0]).at[3, 96:128].set(lg_h_w[0])
    bl = jnp.concatenate([lg_g_b, lg_b_b, lg_w_b, lg_h_b])
    z, d0, h4, lg = _middle(gender_z, en_lin_w, en_lin_b, wh, bh,
                            de_lin_w, de_lin_b, in4, wl, bl)

    d = d0.reshape(_B, _VS[4], 64).transpose(1, 0, 2).reshape(_VS[4], _B * 64)
    dins = (64, 64, 32, 32)
    for j in range(4):
        lvl = 3 - j
        d = _pool(d, dins[j], ucol[lvl], uval[lvl], _VP[lvl])
        co = de_w[j].shape[1]
        d = _conv(d, dins[j], sidx[lvl], de_w[j], de_b[j], True)
        d = d.reshape(_VP[lvl], _B * co)
    out = _conv(d, 32, sidx[0], de_final_w, de_final_b, False)
    out = out.reshape(_VS[0], _B, 3).transpose(1, 0, 2)

    return (
        out,
        h4[:, 3:4],
        lg[:, 32:64],
        lg[:, 0:32],
        lg[:, 64:96],
        lg[:, 96:128],
        z,
        gender_z,
        h4[:, 0:1],
        h4[:, 1:2],
        h4[:, 2:3],
    )


# speedup vs baseline: 3.2087x; 1.0290x over previous
import jax
import jax.numpy as jnp
from jax import lax
from jax.experimental import pallas as pl
from jax.experimental.pallas import tpu as pltpu
from jax.experimental.pallas import tpu_sc as plsc

_B = 16
_VS = (10000, 2500, 625, 157, 40)
_VP = (10000, 2504, 632, 160, 40)
_L = 9
_K = 3
_NC = 2
_NS = 16
_NW = _NC * _NS


def _sc_gather(table, idx_flat):
    N, D = table.shape
    M = idx_flat.shape[0]
    budget_rows = (384 * 1024) // (D * 4)
    sub = min(128, (budget_rows // 8) * 8)
    k_sub = max(1, min(6, budget_rows // sub))
    ch = k_sub * sub
    n_chunks = -(-M // ch)
    pad = n_chunks * ch - M
    if pad:
        idx_flat = jnp.pad(idx_flat, (0, pad))
    idx_nd = idx_flat.reshape(n_chunks, k_sub, sub)
    if k_sub < 8:
        idx_nd = jnp.pad(idx_nd, ((0, 0), (0, 8 - k_sub), (0, 0)))
    idx_scratch = pltpu.VMEM((8, sub), jnp.int32)
    g_full = M // ch
    tail = M - g_full * ch
    n_iter = -(-n_chunks // _NW)
    mesh = plsc.VectorSubcoreMesh(core_axis_name="c", subcore_axis_name="s")

    def body(table_ref, idx_ref, out_ref, idx_v, rows_v, sem):
        wid = lax.axis_index("s") * _NC + lax.axis_index("c")

        def run_chunk(g):
            pltpu.sync_copy(idx_ref.at[g], idx_v)
            cps = [
                pltpu.async_copy(
                    table_ref.at[idx_v.at[j]],
                    rows_v.at[pl.ds(j * sub, sub)],
                    sem,
                )
                for j in range(k_sub)
            ]
            for c in cps:
                c.wait()

        def step(it, carry):
            g = it * _NW + wid
            if g_full:
                @pl.when(g < g_full)
                def _():
                    run_chunk(g)
                    pltpu.sync_copy(rows_v, out_ref.at[pl.ds(g * ch, ch)])
            if tail:
                @pl.when(g == g_full)
                def _():
                    run_chunk(g)
                    pltpu.sync_copy(
                        rows_v.at[pl.ds(0, tail)],
                        out_ref.at[pl.ds(g_full * ch, tail)],
                    )
            return carry

        if n_iter > 1:
            lax.fori_loop(0, n_iter, step, 0)
        else:
            step(0, 0)

    return pl.kernel(
        body,
        out_type=jax.ShapeDtypeStruct((M, D), table.dtype),
        mesh=mesh,
        scratch_types=[
            idx_scratch,
            pltpu.VMEM((ch, D), jnp.float32),
            pltpu.SemaphoreType.DMA,
        ],
    )(table, idx_nd)


def _mm9(g9, w, bias, act):
    _, M, C = g9.shape
    N = w.shape[1]
    bm = 512
    grid = -(-M // bm)

    def body(g_ref, w_ref, b_ref, o_ref):
        acc = b_ref[...].astype(jnp.float32)
        for l in range(_L):
            acc = acc + jnp.dot(
                g_ref[l], w_ref[pl.ds(l * C, C), :],
                preferred_element_type=jnp.float32,
            )
        if act:
            acc = jnp.where(acc > 0, acc, jnp.exp(jnp.minimum(acc, 0.0)) - 1.0)
        o_ref[...] = acc

    return pl.pallas_call(
        body,
        grid=(grid,),
        in_specs=[
            pl.BlockSpec((_L, bm, C), lambda i: (0, i, 0)),
            pl.BlockSpec((_L * C, N), lambda i: (0, 0)),
            pl.BlockSpec((1, N), lambda i: (0, 0)),
        ],
        out_specs=pl.BlockSpec((bm, N), lambda i: (i, 0)),
        out_shape=jax.ShapeDtypeStruct((M, N), jnp.float32),
    )(g9, w, bias.reshape(1, N))


def _wsum(g, w):
    vo, d3 = g.shape
    D = d3 // _K
    bm = 256
    grid = -(-vo // bm)

    def body(g_ref, w_ref, o_ref):
        gg = g_ref[...]
        ww = w_ref[...]
        o_ref[...] = (
            ww[:, 0:1] * gg[:, :D]
            + ww[:, 1:2] * gg[:, D : 2 * D]
            + ww[:, 2:3] * gg[:, 2 * D :]
        )

    return pl.pallas_call(
        body,
        grid=(grid,),
        in_specs=[
            pl.BlockSpec((bm, d3), lambda i: (i, 0)),
            pl.BlockSpec((bm, _K), lambda i: (i, 0)),
        ],
        out_specs=pl.BlockSpec((bm, D), lambda i: (i, 0)),
        out_shape=jax.ShapeDtypeStruct((vo, D), jnp.float32),
    )(g, w)


def _poolmm(p, h):
    vo, vi = p.shape
    D = h.shape[1]
    bm = 512
    grid = -(-vo // bm)

    def body(p_ref, h_ref, o_ref):
        o_ref[...] = jnp.dot(p_ref[...], h_ref[...], preferred_element_type=jnp.float32)

    return pl.pallas_call(
        body,
        grid=(grid,),
        in_specs=[
            pl.BlockSpec((bm, vi), lambda i: (i, 0)),
            pl.BlockSpec((vi, D), lambda i: (0, 0)),
        ],
        out_specs=pl.BlockSpec((bm, D), lambda i: (i, 0)),
        out_shape=jax.ShapeDtypeStruct((vo, D), jnp.float32),
    )(p, h)


def _middle(gz, w1, b1, wh, bh, w2, b2, in4, wl, bl):
    def body(gz_ref, w1_ref, b1_ref, wh_ref, bh_ref, w2_ref, b2_ref,
             i4_ref, wl_ref, bl_ref, z_ref, d0_ref, h4_ref, lg_ref):
        gzv = gz_ref[...]
        z = jnp.dot(gzv, w1_ref[...], preferred_element_type=jnp.float32) + b1_ref[...]
        z_ref[...] = z
        d0_ref[...] = (
            jnp.dot(z, w2_ref[...], preferred_element_type=jnp.float32) + b2_ref[...]
        )
        h4 = jnp.dot(gzv, wh_ref[...], preferred_element_type=jnp.float32) + bh_ref[...]
        is_cl = lax.broadcasted_iota(jnp.int32, h4.shape, 1) == 3
        h4_ref[...] = jnp.where(is_cl, 1.0 / (1.0 + jnp.exp(-h4)), h4)
        lg_ref[...] = (
            jnp.dot(i4_ref[...], wl_ref[...], preferred_element_type=jnp.float32)
            + bl_ref[...]
        )

    nv = gz.shape[1]
    return pl.pallas_call(
        body,
        out_shape=[
            jax.ShapeDtypeStruct((_B, w1.shape[1]), jnp.float32),
            jax.ShapeDtypeStruct((_B, nv), jnp.float32),
            jax.ShapeDtypeStruct((_B, 4), jnp.float32),
            jax.ShapeDtypeStruct((_B, 128), jnp.float32),
        ],
    )(gz, w1, b1.reshape(1, -1), wh, bh.reshape(1, -1), w2, b2.reshape(1, -1),
      in4, wl, bl.reshape(1, -1))


def _conv(h, c, sidx, w, bias, act):
    v = sidx.shape[0]
    g = _sc_gather(h, sidx.T.reshape(-1))
    g9 = g.reshape(_L, v * _B, c)
    return _mm9(g9, w, bias, act)


def _pool(h, c, col, val, vo):
    vi = h.shape[0]
    if vo * _K * vi <= 8 * 1000 * 1000:
        p = (val.reshape(vo, _K, 1)
             * (col.reshape(vo, _K, 1) == jnp.arange(vi, dtype=jnp.int32))).sum(1)
        return _poolmm(p.astype(jnp.float32), h)
    g = _sc_gather(h, col)
    return _wsum(g.reshape(vo, _K * _B * c), val.reshape(vo, _K))


def kernel(x, y, pos, spiral_idx_0, spiral_idx_1, spiral_idx_2, spiral_idx_3, dt0_row, dt0_col, dt0_val, dt1_row, dt1_col, dt1_val, dt2_row, dt2_col, dt2_val, dt3_row, dt3_col, dt3_val, ut0_row, ut0_col, ut0_val, ut1_row, ut1_col, ut1_val, ut2_row, ut2_col, ut2_val, ut3_row, ut3_col, ut3_val, en_w0, en_b0, en_w1, en_b1, en_w2, en_b2, en_w3, en_b3, en_lin_w, en_lin_b, reb_w, reb_b, rew_w, rew_b, reh_w, reh_b, cl_w, cl_b, lg_g_w, lg_g_b, lg_b_w, lg_b_b, lg_w_w, lg_w_b, lg_h_w, lg_h_b, de_lin_w, de_lin_b, de_w0, de_b0, de_w1, de_b1, de_w2, de_b2, de_w3, de_b3, de_final_w, de_final_b):
    def padv(a, n):
        return jnp.pad(a, ((0, n - a.shape[0]),) + ((0, 0),) * (a.ndim - 1))

    sidx = tuple(padv(s, _VP[i])
                 for i, s in enumerate((spiral_idx_0, spiral_idx_1, spiral_idx_2, spiral_idx_3)))
    dcol = tuple(padv(c, _VP[i + 1] * _K)
                 for i, c in enumerate((dt0_col, dt1_col, dt2_col, dt3_col)))
    dval = tuple(padv(v, _VP[i + 1] * _K)
                 for i, v in enumerate((dt0_val, dt1_val, dt2_val, dt3_val)))
    ucol = tuple(padv(c, _VP[i] * _K)
                 for i, c in enumerate((ut0_col, ut1_col, ut2_col, ut3_col)))
    uval = tuple(padv(v, _VP[i] * _K)
                 for i, v in enumerate((ut0_val, ut1_val, ut2_val, ut3_val)))
    en_w = (en_w0, en_w1, en_w2, en_w3)
    en_b = (en_b0, en_b1, en_b2, en_b3)
    de_w = (de_w0, de_w1, de_w2, de_w3)
    de_b = (de_b0, de_b1, de_b2, de_b3)
    chans = (8, 32, 32, 32)

    h = jnp.pad(x.transpose(1, 0, 2), ((0, 0), (0, 0), (0, 5))).reshape(_VS[0], _B * 8)
    w0 = jnp.pad(en_w0.reshape(_L, 3, -1), ((0, 0), (0, 5), (0, 0))).reshape(_L * 8, -1)
    for i in range(4):
        wi = w0 if i == 0 else en_w[i]
        co = en_w[i].shape[1]
        h = _conv(h, chans[i], sidx[i], wi, en_b[i], True)
        h = _pool(h.reshape(_VP[i], _B * co), co, dcol[i], dval[i], _VP[i + 1])

    gender_z = h.reshape(_VS[4], _B, 64).transpose(1, 0, 2).reshape(_B, _VS[4] * 64)

    wh = jnp.concatenate([reb_w, rew_w, reh_w, cl_w], axis=1)
    bh = jnp.concatenate([reb_b, rew_b, reh_b, cl_b])
    in4 = jnp.concatenate([y, pos[:, 0:1], pos[:, 1:2], pos[:, 2:3]], axis=1)
    wl = jnp.zeros((4, 128), jnp.float32)
    wl = wl.at[0, 0:32].set(lg_g_w[0]).at[1, 32:64].set(lg_b_w[0])
    wl = wl.at[2, 64:96].set(lg_w_w[0]).at[3, 96:128].set(lg_h_w[0])
    bl = jnp.concatenate([lg_g_b, lg_b_b, lg_w_b, lg_h_b])
    z, d0, h4, lg = _middle(gender_z, en_lin_w, en_lin_b, wh, bh,
                            de_lin_w, de_lin_b, in4, wl, bl)

    d = d0.reshape(_B, _VS[4], 64).transpose(1, 0, 2).reshape(_VS[4], _B * 64)
    dins = (64, 64, 32, 32)
    for j in range(4):
        lvl = 3 - j
        d = _pool(d, dins[j], ucol[lvl], uval[lvl], _VP[lvl])
        co = de_w[j].shape[1]
        d = _conv(d, dins[j], sidx[lvl], de_w[j], de_b[j], True)
        d = d.reshape(_VP[lvl], _B * co)
    out = _conv(d, 32, sidx[0], de_final_w, de_final_b, False)
    out = out.reshape(_VS[0], _B, 3).transpose(1, 0, 2)

    return (
        out,
        h4[:, 3:4],
        lg[:, 32:64],
        lg[:, 0:32],
        lg[:, 64:96],
        lg[:, 96:128],
        z,
        gender_z,
        h4[:, 0:1],
        h4[:, 1:2],
        h4[:, 2:3],
    )
